# baseline (speedup 1.0000x reference)
import functools

import jax
import jax.numpy as jnp
from jax import lax
from jax.experimental import pallas as pl
from jax.experimental.pallas import tpu as pltpu
from jax.experimental.pallas import tpu_sc as plsc

N = 10000
E = 160000
F = 128
S = 16

f32 = jnp.float32

_NH = N // 2
_NHP = _NH + 8
_ACR = _NHP // 8
_TW = 384


def _lane_bcast(v, s):
    idx = jnp.full((16, 1), s, dtype=jnp.int32)
    return lax.gather(
        v, idx,
        lax.GatherDimensionNumbers(
            offset_dims=(), collapsed_slice_dims=(0,), start_index_map=(0,)),
        (1,), mode=lax.GatherScatterMode.PROMISE_IN_BOUNDS)


@functools.lru_cache(None)
def _make_sc_edge_call(NJ, NR, EP):
    C = 32
    NROLE = 2 * NJ
    NT = NROLE * NR
    ESL = EP // NR
    NCH = ESL // C

    mesh = plsc.VectorSubcoreMesh(core_axis_name="c", subcore_axis_name="s")

    def body(hs_hbm, srcj_hbm, tl_hbm, e_hbm, out_hbm,
             idx_v, tloc_v, ew_v, rows_v, acc, sem):
        cid = lax.axis_index("c")
        sid = lax.axis_index("s")
        wid = jnp.minimum(cid * 16 + sid, NT - 1)
        role = wid // NR
        rep = wid % NR
        h = role // NJ
        j = role % NJ

        zero16 = jnp.zeros((16,), f32)
        iota16 = jnp.arange(16, dtype=jnp.int32)

        def zrow(r, carry):
            acc[pl.ds(16 * r, 16)] = zero16
            return carry
        lax.fori_loop(0, _ACR * 8, zrow, 0)

        base_e = rep * ESL
        src_off = j * EP
        tl_off = h * EP

        def chunk(kk, carry):
            b = base_e + kk * C
            pltpu.sync_copy(srcj_hbm.at[pl.ds(src_off + b, C)], idx_v)
            pltpu.sync_copy(tl_hbm.at[pl.ds(tl_off + b, C)], tloc_v)
            pltpu.sync_copy(e_hbm.at[pl.ds(b * 16, C * 16)], ew_v)
            pltpu.async_copy(hs_hbm.at[idx_v], rows_v, sem).wait()

            def group(g, inner):
                tlvec = tloc_v[pl.ds(16 * g, 16)]
                for u in range(16):
                    c = 16 * g + u
                    ev = ew_v[pl.ds(16 * c, 16)]
                    m = rows_v[c, pl.ds(256, 16)]
                    for s in range(S):
                        m = m + _lane_bcast(ev, s) * rows_v[c, pl.ds(16 * s, 16)]
                    tlb = _lane_bcast(tlvec, u)
                    plsc.addupdate_scatter(acc, [tlb * 16 + iota16], m)
                return inner
            lax.fori_loop(0, C // 16, group, 0)
            return carry
        lax.fori_loop(0, NCH, chunk, 0)

        pltpu.sync_copy(acc, out_hbm.at[wid])

    return pl.kernel(
        body,
        out_type=jax.ShapeDtypeStruct((NT, _ACR * 128), f32),
        mesh=mesh,
        compiler_params=pltpu.CompilerParams(needs_layout_passes=False),
        scratch_types=[
            pltpu.VMEM((C,), jnp.int32),
            pltpu.VMEM((C,), jnp.int32),
            pltpu.VMEM((C * 16,), f32),
            pltpu.VMEM((C, _TW), f32),
            pltpu.VMEM((_ACR * 128,), f32),
            pltpu.SemaphoreType.DMA,
        ],
    )


_RB = 1000
_NBH = _NH // _RB


def _mm_body(x_ref, w_ref, o_ref):
    o_ref[...] = jnp.dot(x_ref[...], w_ref[...], preferred_element_type=f32)


def _tc_project(x, W):
    K = W.shape[1]
    Fin = W.shape[0]
    return pl.pallas_call(
        _mm_body,
        grid=(N // _RB,),
        in_specs=[pl.BlockSpec((_RB, Fin), lambda i: (i, 0)),
                  pl.BlockSpec((Fin, K), lambda i: (0, 0))],
        out_specs=pl.BlockSpec((_RB, K), lambda i: (i, 0)),
        out_shape=jax.ShapeDtypeStruct((N, K), f32),
    )(x, W)


def _p_specs(NJ, NR):
    specs = []
    for j in range(NJ):
        for r in range(NR):
            def imap(i, j=j, r=r, NJ=NJ, NR=NR):
                return ((i // _NBH * NJ + j) * NR + r, i % _NBH, 0)
            specs.append(pl.BlockSpec((1, _RB, 16), imap))
    return specs


def _agg_from_views(p_refs, NJ, NR):
    chunks = []
    for j in range(NJ):
        t = p_refs[j * NR][0]
        for r in range(1, NR):
            t = t + p_refs[j * NR + r][0]
        chunks.append(t)
    return jnp.concatenate(chunks, axis=1)


def _tc_finalize_project(P, NJ, NR, x, Rp, rbp, Wnext):
    Fpad = 16 * NJ
    Fin = x.shape[1]
    K = Wnext.shape[1]
    nv = NJ * NR

    def bodyfn(*refs):
        p_refs = refs[:nv]
        x_ref, r_ref, rb_ref, w_ref, h_ref, o_ref = refs[nv:]
        agg = _agg_from_views(p_refs, NJ, NR)
        hh = jnp.maximum(
            agg + jnp.dot(x_ref[...], r_ref[...], preferred_element_type=f32)
            + rb_ref[...], 0.0)
        h_ref[...] = hh
        o_ref[...] = jnp.dot(hh, w_ref[...], preferred_element_type=f32)

    return pl.pallas_call(
        bodyfn,
        grid=(N // _RB,),
        in_specs=_p_specs(NJ, NR)
                 + [pl.BlockSpec((_RB, Fin), lambda i: (i, 0)),
                    pl.BlockSpec((Fin, Fpad), lambda i: (0, 0)),
                    pl.BlockSpec((1, Fpad), lambda i: (0, 0)),
                    pl.BlockSpec((Fpad, K), lambda i: (0, 0))],
        out_specs=[pl.BlockSpec((_RB, Fpad), lambda i: (i, 0)),
                   pl.BlockSpec((_RB, K), lambda i: (i, 0))],
        out_shape=[jax.ShapeDtypeStruct((N, Fpad), f32),
                   jax.ShapeDtypeStruct((N, K), f32)],
    )(*([P] * nv), x, Rp, rbp, Wnext)


def _tc_final(P, NJ, NR, x, Rp, rbp, mlp):
    Fpad = 16 * NJ
    Fin = x.shape[1]
    nv = NJ * NR
    NG = N // _RB

    def wspec(a):
        return pl.BlockSpec(a.shape, lambda i: (0,) * a.ndim)

    def bodyfn(*refs):
        p_refs = refs[:nv]
        x_ref, r_ref, rb_ref = refs[nv:nv + 3]
        w1, b1, w2, b2, w3, b3, w5, b5, w6, b6, w8, b8 = refs[nv + 3:nv + 15]
        o_ref = refs[nv + 15]
        acc_ref = refs[nv + 16]
        i = pl.program_id(0)
        agg = _agg_from_views(p_refs, NJ, NR)
        hh = jnp.maximum(
            agg + jnp.dot(x_ref[...], r_ref[...], preferred_element_type=f32)
            + rb_ref[...], 0.0)
        part = jnp.sum(hh, axis=0, keepdims=True)

        @pl.when(i == 0)
        def _():
            acc_ref[...] = part

        @pl.when(i > 0)
        def _():
            acc_ref[...] = acc_ref[...] + part

        @pl.when(i == NG - 1)
        def _():
            z = acc_ref[...]
            z = jnp.maximum(jnp.dot(z, w1[...], preferred_element_type=f32) + b1[...], 0.0)
            z = jnp.maximum(jnp.dot(z, w2[...], preferred_element_type=f32) + b2[...], 0.0)
            z = jnp.maximum(jnp.dot(z, w3[...], preferred_element_type=f32) + b3[...], 0.0)
            z = jnp.maximum(jnp.dot(z, w5[...], preferred_element_type=f32) + b5[...], 0.0)
            z = jnp.maximum(jnp.dot(z, w6[...], preferred_element_type=f32) + b6[...], 0.0)
            o_ref[...] = jnp.dot(z, w8[...], preferred_element_type=f32) + b8[...]

    return pl.pallas_call(
        bodyfn,
        grid=(NG,),
        in_specs=_p_specs(NJ, NR)
                 + [pl.BlockSpec((_RB, Fin), lambda i: (i, 0)),
                    pl.BlockSpec((Fin, Fpad), lambda i: (0, 0)),
                    pl.BlockSpec((1, Fpad), lambda i: (0, 0))]
                 + [wspec(a) for a in mlp],
        out_specs=pl.BlockSpec((1, 1), lambda i: (0, 0)),
        out_shape=jax.ShapeDtypeStruct((1, 1), f32),
        scratch_shapes=[pltpu.VMEM((1, Fpad), f32)],
    )(*([P] * nv), x, Rp, rbp, *mlp)


def _pad2(A, r, c):
    return jnp.pad(A, ((0, r - A.shape[0]), (0, c - A.shape[1])))


def _build_W2J(Wk, bk, Fin, Fout, NJ, Fin_pad):
    Wk_r = Wk.reshape(S, Fin, Fout)
    bk_r = bk.reshape(Fin, Fout)
    cols = []
    for j in range(NJ):
        lo = 16 * j
        hi = min(16 * j + 16, Fout)
        for s in range(S):
            cols.append(_pad2(Wk_r[s][:, lo:hi], Fin, 16))
        cols.append(_pad2(bk_r[:, lo:hi], Fin, 16))
        cols.append(jnp.zeros((Fin, _TW - 17 * 16), f32))
    return _pad2(jnp.concatenate(cols, axis=1), Fin_pad, NJ * _TW)


def kernel(x, edge_index, e, Wk1, bk1, R1, rb1, Wk2, bk2, R2, rb2,
           Wk3, bk3, R3, rb3, Wd1, bd1, Wd2, bd2, Wd3, bd3, Wd5, bd5,
           Wd6, bd6, Wd8, bd8):
    src = edge_index[0]
    tgt = edge_index[1]
    ef = e.reshape(-1)
    tl0 = jnp.where(tgt < _NH, tgt, _NH)
    tl1 = jnp.where(tgt >= _NH, tgt - _NH, _NH)
    tl = jnp.concatenate([tl0, tl1])

    NJ1, NR1 = 3, 2
    NJ23, NR23 = 2, 3
    EP1 = E
    EP23 = 160032
    PAD = EP23 - E
    FP1, FP2, FP3 = 48, 32, 32

    W21 = _build_W2J(Wk1, bk1, F, 40, NJ1, F)
    W22 = _build_W2J(Wk2, bk2, 40, 24, NJ23, FP1)
    W23 = _build_W2J(Wk3, bk3, 24, 24, NJ23, FP2)
    src3 = jnp.concatenate([src, src + N, src + 2 * N])
    zpad = jnp.zeros((PAD,), jnp.int32)
    srcp = jnp.concatenate([src, zpad])
    src2 = jnp.concatenate([srcp, srcp + N])
    R1p = _pad2(R1, F, FP1)
    R2p = _pad2(R2, FP1, FP2)
    R3p = _pad2(R3, FP2, FP3)
    rb1p = _pad2(rb1.reshape(1, -1), 1, FP1)
    rb2p = _pad2(rb2.reshape(1, -1), 1, FP2)
    rb3p = _pad2(rb3.reshape(1, -1), 1, FP3)
    Wd1p = _pad2(Wd1, FP3, Wd1.shape[1])
    mlp = (Wd1p, bd1.reshape(1, -1), Wd2, bd2.reshape(1, -1),
           Wd3, bd3.reshape(1, -1), Wd5, bd5.reshape(1, -1),
           Wd6, bd6.reshape(1, -1), Wd8, bd8.reshape(1, -1))

    dpad = jnp.full((PAD,), _NH, jnp.int32)
    tlp = jnp.concatenate([tl0, dpad, tl1, dpad])
    efp = jnp.concatenate([ef, jnp.zeros((PAD * 16,), f32)])

    sc1 = _make_sc_edge_call(NJ1, NR1, EP1)
    sc23 = _make_sc_edge_call(NJ23, NR23, EP23)

    def stack_tables(Hflat, NJ):
        return Hflat.reshape(N, NJ, _TW).transpose(1, 0, 2).reshape(NJ * N, _TW)

    H1 = stack_tables(_tc_project(x, W21), NJ1)
    P1 = sc1(H1, src3, tl, ef).reshape(-1, _NHP, 16)
    h1, H2f = _tc_finalize_project(P1, NJ1, NR1, x, R1p, rb1p, W22)
    H2 = stack_tables(H2f, NJ23)
    P2 = sc23(H2, src2, tlp, efp).reshape(-1, _NHP, 16)
    h2, H3f = _tc_finalize_project(P2, NJ23, NR23, h1, R2p, rb2p, W23)
    H3 = stack_tables(H3f, NJ23)
    P3 = sc23(H3, src2, tlp, efp).reshape(-1, _NHP, 16)
    return _tc_final(P3, NJ23, NR23, h2, R3p, rb3p, mlp)


# speedup vs baseline: 1.3994x; 1.3994x over previous
import functools

import jax
import jax.numpy as jnp
from jax import lax
from jax.experimental import pallas as pl
from jax.experimental.pallas import tpu as pltpu
from jax.experimental.pallas import tpu_sc as plsc

N = 10000
E = 160000
F = 128
S = 16

f32 = jnp.float32

_NH = N // 2
_NHP = _NH + 8
_ACR = _NHP // 8
_TW = 384


def _lane_bcast(v, s):
    idx = jnp.full((16, 1), s, dtype=jnp.int32)
    return lax.gather(
        v, idx,
        lax.GatherDimensionNumbers(
            offset_dims=(), collapsed_slice_dims=(0,), start_index_map=(0,)),
        (1,), mode=lax.GatherScatterMode.PROMISE_IN_BOUNDS)


@functools.lru_cache(None)
def _make_sc_edge_call(NJ, NR, EP):
    C = 64
    NROLE = 2 * NJ
    NT = NROLE * NR
    ESL = EP // NR
    NCH = ESL // C

    mesh = plsc.VectorSubcoreMesh(core_axis_name="c", subcore_axis_name="s")

    def body(hs_hbm, srcj_hbm, tl_hbm, e_hbm, out_hbm,
             idx_v, tloc_v, ew_v, rows_v, acc, sem):
        cid = lax.axis_index("c")
        sid = lax.axis_index("s")
        wid = jnp.minimum(cid * 16 + sid, NT - 1)
        role = wid // NR
        rep = wid % NR
        h = role // NJ
        j = role % NJ

        zero16 = jnp.zeros((16,), f32)
        iota16 = jnp.arange(16, dtype=jnp.int32)

        def zrow(r, carry):
            acc[pl.ds(16 * r, 16)] = zero16
            return carry
        lax.fori_loop(0, _ACR * 8, zrow, 0)

        base_e = rep * ESL
        src_off = j * EP
        tl_off = h * EP

        def chunk(kk, carry):
            b = base_e + kk * C
            pltpu.sync_copy(srcj_hbm.at[pl.ds(src_off + b, C)], idx_v)
            pltpu.sync_copy(tl_hbm.at[pl.ds(tl_off + b, C)], tloc_v)
            pltpu.sync_copy(e_hbm.at[pl.ds(b * 16, C * 16)], ew_v)
            pltpu.async_copy(hs_hbm.at[idx_v], rows_v, sem).wait()

            def group(g, inner):
                tlvec = tloc_v[pl.ds(16 * g, 16)]
                for u in range(16):
                    c = 16 * g + u
                    ev = ew_v[pl.ds(16 * c, 16)]
                    m = rows_v[c, pl.ds(256, 16)]
                    for s in range(S):
                        m = m + _lane_bcast(ev, s) * rows_v[c, pl.ds(16 * s, 16)]
                    tlb = _lane_bcast(tlvec, u)
                    plsc.addupdate_scatter(acc, [tlb * 16 + iota16], m)
                return inner
            lax.fori_loop(0, C // 16, group, 0)
            return carry
        lax.fori_loop(0, NCH, chunk, 0)

        pltpu.sync_copy(acc, out_hbm.at[wid])

    return pl.kernel(
        body,
        out_type=jax.ShapeDtypeStruct((NT, _ACR * 128), f32),
        mesh=mesh,
        compiler_params=pltpu.CompilerParams(needs_layout_passes=False),
        scratch_types=[
            pltpu.VMEM((C,), jnp.int32),
            pltpu.VMEM((C,), jnp.int32),
            pltpu.VMEM((C * 16,), f32),
            pltpu.VMEM((C, _TW), f32),
            pltpu.VMEM((_ACR * 128,), f32),
            pltpu.SemaphoreType.DMA,
        ],
    )


_RB = 1000
_NBH = _NH // _RB


def _mm_body(x_ref, w_ref, o_ref):
    o_ref[...] = jnp.dot(x_ref[...], w_ref[...], preferred_element_type=f32)


def _tc_project(x, W):
    K = W.shape[1]
    Fin = W.shape[0]
    return pl.pallas_call(
        _mm_body,
        grid=(N // _RB,),
        in_specs=[pl.BlockSpec((_RB, Fin), lambda i: (i, 0)),
                  pl.BlockSpec((Fin, K), lambda i: (0, 0))],
        out_specs=pl.BlockSpec((_RB, K), lambda i: (i, 0)),
        out_shape=jax.ShapeDtypeStruct((N, K), f32),
    )(x, W)


def _p_specs(NJ, NR):
    specs = []
    for j in range(NJ):
        for r in range(NR):
            def imap(i, j=j, r=r, NJ=NJ, NR=NR):
                return ((i // _NBH * NJ + j) * NR + r, i % _NBH, 0)
            specs.append(pl.BlockSpec((1, _RB, 16), imap))
    return specs


def _agg_from_views(p_refs, NJ, NR):
    chunks = []
    for j in range(NJ):
        t = p_refs[j * NR][0]
        for r in range(1, NR):
            t = t + p_refs[j * NR + r][0]
        chunks.append(t)
    return jnp.concatenate(chunks, axis=1)


def _tc_finalize_project(P, NJ, NR, x, Rp, rbp, Wnext):
    Fpad = 16 * NJ
    Fin = x.shape[1]
    K = Wnext.shape[1]
    nv = NJ * NR

    def bodyfn(*refs):
        p_refs = refs[:nv]
        x_ref, r_ref, rb_ref, w_ref, h_ref, o_ref = refs[nv:]
        agg = _agg_from_views(p_refs, NJ, NR)
        hh = jnp.maximum(
            agg + jnp.dot(x_ref[...], r_ref[...], preferred_element_type=f32)
            + rb_ref[...], 0.0)
        h_ref[...] = hh
        o_ref[...] = jnp.dot(hh, w_ref[...], preferred_element_type=f32)

    return pl.pallas_call(
        bodyfn,
        grid=(N // _RB,),
        in_specs=_p_specs(NJ, NR)
                 + [pl.BlockSpec((_RB, Fin), lambda i: (i, 0)),
                    pl.BlockSpec((Fin, Fpad), lambda i: (0, 0)),
                    pl.BlockSpec((1, Fpad), lambda i: (0, 0)),
                    pl.BlockSpec((Fpad, K), lambda i: (0, 0))],
        out_specs=[pl.BlockSpec((_RB, Fpad), lambda i: (i, 0)),
                   pl.BlockSpec((_RB, K), lambda i: (i, 0))],
        out_shape=[jax.ShapeDtypeStruct((N, Fpad), f32),
                   jax.ShapeDtypeStruct((N, K), f32)],
    )(*([P] * nv), x, Rp, rbp, Wnext)


def _tc_final(P, NJ, NR, x, Rp, rbp, mlp):
    Fpad = 16 * NJ
    Fin = x.shape[1]
    nv = NJ * NR
    NG = N // _RB

    def wspec(a):
        return pl.BlockSpec(a.shape, lambda i: (0,) * a.ndim)

    def bodyfn(*refs):
        p_refs = refs[:nv]
        x_ref, r_ref, rb_ref = refs[nv:nv + 3]
        w1, b1, w2, b2, w3, b3, w5, b5, w6, b6, w8, b8 = refs[nv + 3:nv + 15]
        o_ref = refs[nv + 15]
        acc_ref = refs[nv + 16]
        i = pl.program_id(0)
        agg = _agg_from_views(p_refs, NJ, NR)
        hh = jnp.maximum(
            agg + jnp.dot(x_ref[...], r_ref[...], preferred_element_type=f32)
            + rb_ref[...], 0.0)
        part = jnp.sum(hh, axis=0, keepdims=True)

        @pl.when(i == 0)
        def _():
            acc_ref[...] = part

        @pl.when(i > 0)
        def _():
            acc_ref[...] = acc_ref[...] + part

        @pl.when(i == NG - 1)
        def _():
            z = acc_ref[...]
            z = jnp.maximum(jnp.dot(z, w1[...], preferred_element_type=f32) + b1[...], 0.0)
            z = jnp.maximum(jnp.dot(z, w2[...], preferred_element_type=f32) + b2[...], 0.0)
            z = jnp.maximum(jnp.dot(z, w3[...], preferred_element_type=f32) + b3[...], 0.0)
            z = jnp.maximum(jnp.dot(z, w5[...], preferred_element_type=f32) + b5[...], 0.0)
            z = jnp.maximum(jnp.dot(z, w6[...], preferred_element_type=f32) + b6[...], 0.0)
            o_ref[...] = jnp.dot(z, w8[...], preferred_element_type=f32) + b8[...]

    return pl.pallas_call(
        bodyfn,
        grid=(NG,),
        in_specs=_p_specs(NJ, NR)
                 + [pl.BlockSpec((_RB, Fin), lambda i: (i, 0)),
                    pl.BlockSpec((Fin, Fpad), lambda i: (0, 0)),
                    pl.BlockSpec((1, Fpad), lambda i: (0, 0))]
                 + [wspec(a) for a in mlp],
        out_specs=pl.BlockSpec((1, 1), lambda i: (0, 0)),
        out_shape=jax.ShapeDtypeStruct((1, 1), f32),
        scratch_shapes=[pltpu.VMEM((1, Fpad), f32)],
    )(*([P] * nv), x, Rp, rbp, *mlp)


def _pad2(A, r, c):
    return jnp.pad(A, ((0, r - A.shape[0]), (0, c - A.shape[1])))


def _build_W2J(Wk, bk, Fin, Fout, NJ, Fin_pad):
    Wk_r = Wk.reshape(S, Fin, Fout)
    bk_r = bk.reshape(Fin, Fout)
    cols = []
    for j in range(NJ):
        lo = 16 * j
        hi = min(16 * j + 16, Fout)
        for s in range(S):
            cols.append(_pad2(Wk_r[s][:, lo:hi], Fin, 16))
        cols.append(_pad2(bk_r[:, lo:hi], Fin, 16))
        cols.append(jnp.zeros((Fin, _TW - 17 * 16), f32))
    return _pad2(jnp.concatenate(cols, axis=1), Fin_pad, NJ * _TW)


def kernel(x, edge_index, e, Wk1, bk1, R1, rb1, Wk2, bk2, R2, rb2,
           Wk3, bk3, R3, rb3, Wd1, bd1, Wd2, bd2, Wd3, bd3, Wd5, bd5,
           Wd6, bd6, Wd8, bd8):
    src = edge_index[0]
    tgt = edge_index[1]
    ef = e.reshape(-1)
    tl0 = jnp.where(tgt < _NH, tgt, _NH)
    tl1 = jnp.where(tgt >= _NH, tgt - _NH, _NH)
    tl = jnp.concatenate([tl0, tl1])

    NJ1, NR1 = 3, 2
    NJ23, NR23 = 2, 3
    EP1 = E
    EP23 = 160128
    PAD = EP23 - E
    FP1, FP2, FP3 = 48, 32, 32

    W21 = _build_W2J(Wk1, bk1, F, 40, NJ1, F)
    W22 = _build_W2J(Wk2, bk2, 40, 24, NJ23, FP1)
    W23 = _build_W2J(Wk3, bk3, 24, 24, NJ23, FP2)
    src3 = jnp.concatenate([src, src + N, src + 2 * N])
    zpad = jnp.zeros((PAD,), jnp.int32)
    srcp = jnp.concatenate([src, zpad])
    src2 = jnp.concatenate([srcp, srcp + N])
    R1p = _pad2(R1, F, FP1)
    R2p = _pad2(R2, FP1, FP2)
    R3p = _pad2(R3, FP2, FP3)
    rb1p = _pad2(rb1.reshape(1, -1), 1, FP1)
    rb2p = _pad2(rb2.reshape(1, -1), 1, FP2)
    rb3p = _pad2(rb3.reshape(1, -1), 1, FP3)
    Wd1p = _pad2(Wd1, FP3, Wd1.shape[1])
    mlp = (Wd1p, bd1.reshape(1, -1), Wd2, bd2.reshape(1, -1),
           Wd3, bd3.reshape(1, -1), Wd5, bd5.reshape(1, -1),
           Wd6, bd6.reshape(1, -1), Wd8, bd8.reshape(1, -1))

    dpad = jnp.full((PAD,), _NH, jnp.int32)
    tlp = jnp.concatenate([tl0, dpad, tl1, dpad])
    efp = jnp.concatenate([ef, jnp.zeros((PAD * 16,), f32)])

    sc1 = _make_sc_edge_call(NJ1, NR1, EP1)
    sc23 = _make_sc_edge_call(NJ23, NR23, EP23)

    def stack_tables(Hflat, NJ):
        return Hflat.reshape(N, NJ, _TW).transpose(1, 0, 2).reshape(NJ * N, _TW)

    H1 = stack_tables(_tc_project(x, W21), NJ1)
    P1 = sc1(H1, src3, tl, ef).reshape(-1, _NHP, 16)
    h1, H2f = _tc_finalize_project(P1, NJ1, NR1, x, R1p, rb1p, W22)
    H2 = stack_tables(H2f, NJ23)
    P2 = sc23(H2, src2, tlp, efp).reshape(-1, _NHP, 16)
    h2, H3f = _tc_finalize_project(P2, NJ23, NR23, h1, R2p, rb2p, W23)
    H3 = stack_tables(H3f, NJ23)
    P3 = sc23(H3, src2, tlp, efp).reshape(-1, _NHP, 16)
    return _tc_final(P3, NJ23, NR23, h2, R3p, rb3p, mlp)
